# Initial kernel scaffold; baseline (speedup 1.0000x reference)
import jax
import jax.numpy as jnp
from jax.experimental import pallas as pl


def kernel(x, edge_index, edge_weight, W1, b1, W2, b2):
    raise NotImplementedError("write your pallas kernel here")


# speedup vs baseline: 9.8523x; 9.8523x over previous
import functools

import jax
import jax.numpy as jnp
from jax import lax
from jax.experimental import pallas as pl
from jax.experimental.pallas import tpu as pltpu
from jax.experimental.pallas import tpu_sc as plsc

NC = 2
NS = 16
NW = NC * NS
LANES = 16


def _make_deg_kernel(n, e):
    assert e % NW == 0
    e_per = e // NW
    assert e_per % LANES == 0 and n % LANES == 0
    mesh = plsc.VectorSubcoreMesh(core_axis_name="c", subcore_axis_name="s")

    @functools.partial(
        pl.kernel,
        mesh=mesh,
        compiler_params=pltpu.CompilerParams(needs_layout_passes=False),
        out_type=jax.ShapeDtypeStruct((NW * n,), jnp.float32),
        scratch_types=[
            pltpu.VMEM((e_per,), jnp.int32),
            pltpu.VMEM((e_per,), jnp.float32),
            pltpu.VMEM((n,), jnp.float32),
        ],
    )
    def deg_kernel(dst_hbm, ew_hbm, out_hbm, dst_v, ew_v, deg_v):
        c = lax.axis_index("c")
        s = lax.axis_index("s")
        wid = s * NC + c
        base = pl.multiple_of(wid * e_per, 8)
        pltpu.sync_copy(dst_hbm.at[pl.ds(base, e_per)], dst_v)
        pltpu.sync_copy(ew_hbm.at[pl.ds(base, e_per)], ew_v)

        zeros = jnp.zeros((LANES,), jnp.float32)

        def zero_body(i, carry):
            deg_v[pl.ds(i * LANES, LANES)] = zeros
            return carry

        lax.fori_loop(0, n // LANES, zero_body, 0)

        def edge_body(g, carry):
            idx = dst_v[pl.ds(g * LANES, LANES)]
            w = ew_v[pl.ds(g * LANES, LANES)]
            plsc.addupdate_scatter(deg_v, [idx], w)
            return carry

        lax.fori_loop(0, e_per // LANES, edge_body, 0)
        out0 = pl.multiple_of(wid * n, 8)
        pltpu.sync_copy(deg_v, out_hbm.at[pl.ds(out0, n)])

    return deg_kernel


_CHUNK = 80


def _make_agg_kernel(n, e, d):
    assert e % NW == 0
    e_per = e // NW
    assert e_per % _CHUNK == 0
    n_chunks = e_per // _CHUNK
    mesh = plsc.VectorSubcoreMesh(core_axis_name="c", subcore_axis_name="s")

    @functools.partial(
        pl.kernel,
        mesh=mesh,
        compiler_params=pltpu.CompilerParams(needs_layout_passes=False),
        out_type=jax.ShapeDtypeStruct((NC, n, d), jnp.float32),
        scratch_types=[
            pltpu.VMEM((_CHUNK,), jnp.int32),
            pltpu.VMEM((_CHUNK,), jnp.int32),
            pltpu.VMEM((_CHUNK,), jnp.float32),
            pltpu.VMEM((_CHUNK, d), jnp.float32),
            pltpu.VMEM_SHARED((n, d), jnp.float32),
        ],
    )
    def agg_kernel(hp_hbm, src_hbm, dst_hbm, ew_hbm, zeros_hbm, out_hbm,
                   src_v, dst_v, ew_v, rows_v, acc_sh):
        c = lax.axis_index("c")
        s = lax.axis_index("s")
        wid = s * NC + c
        base = pl.multiple_of(wid * e_per, 8)

        @pl.when(s == 0)
        def _():
            pltpu.sync_copy(zeros_hbm, acc_sh)

        plsc.subcore_barrier()

        def chunk_body(g, carry):
            b0 = pl.multiple_of(base + g * _CHUNK, 8)
            pltpu.sync_copy(src_hbm.at[pl.ds(b0, _CHUNK)], src_v)
            pltpu.sync_copy(dst_hbm.at[pl.ds(b0, _CHUNK)], dst_v)
            pltpu.sync_copy(ew_hbm.at[pl.ds(b0, _CHUNK)], ew_v)
            pltpu.sync_copy(hp_hbm.at[src_v], rows_v)

            def row_body(i, carry2):
                w = plsc.load_gather(ew_v, [jnp.full((LANES,), i, jnp.int32)])
                for j in range(d // LANES):
                    sl = pl.ds(j * LANES, LANES)
                    rows_v[i, sl] = rows_v[i, sl] * w
                return carry2

            lax.fori_loop(0, _CHUNK, row_body, 0)
            pltpu.sync_copy(rows_v, acc_sh.at[dst_v], add=True)
            return carry

        lax.fori_loop(0, n_chunks, chunk_body, 0)
        plsc.subcore_barrier()

        @pl.when(s == 0)
        def _():
            pltpu.sync_copy(acc_sh, out_hbm.at[c])

    return agg_kernel


_ROWS = 1000


def _tc_a_body(parts_ref, x_ref, w_ref, dis_ref, hp_ref):
    deg = jnp.sum(parts_ref[...], axis=1, keepdims=True) + 1.0
    dis = lax.rsqrt(deg)
    dis_ref[...] = dis
    h = jnp.dot(x_ref[...], w_ref[...], preferred_element_type=jnp.float32)
    hp_ref[...] = dis * h


def _tc_b_body(p_ref, hp_ref, dis_ref, b_ref, w_ref, hp2_ref):
    dis = dis_ref[...]
    agg = p_ref[0] + p_ref[1] + hp_ref[...]
    z = jnp.maximum(dis * agg + b_ref[...], 0.0)
    h2 = jnp.dot(z, w_ref[...], preferred_element_type=jnp.float32)
    hp2_ref[...] = dis * h2


def _tc_c_body(q_ref, hp2_ref, dis_ref, b_ref, out_ref):
    dis = dis_ref[...]
    out_ref[...] = dis * (q_ref[0] + q_ref[1] + hp2_ref[...]) + b_ref[...]


def _tc_a(parts, x, w1, n, d):
    grid = (n // _ROWS,)
    return pl.pallas_call(
        _tc_a_body,
        grid=grid,
        in_specs=[
            pl.BlockSpec((_ROWS, NW), lambda i: (i, 0)),
            pl.BlockSpec((_ROWS, d), lambda i: (i, 0)),
            pl.BlockSpec((d, d), lambda i: (0, 0)),
        ],
        out_specs=[
            pl.BlockSpec((_ROWS, 1), lambda i: (i, 0)),
            pl.BlockSpec((_ROWS, d), lambda i: (i, 0)),
        ],
        out_shape=[
            jax.ShapeDtypeStruct((n, 1), jnp.float32),
            jax.ShapeDtypeStruct((n, d), jnp.float32),
        ],
    )(parts, x, w1)


def _tc_b(p, hp, dis, b1, w2, n, d):
    grid = (n // _ROWS,)
    return pl.pallas_call(
        _tc_b_body,
        grid=grid,
        in_specs=[
            pl.BlockSpec((NC, _ROWS, d), lambda i: (0, i, 0)),
            pl.BlockSpec((_ROWS, d), lambda i: (i, 0)),
            pl.BlockSpec((_ROWS, 1), lambda i: (i, 0)),
            pl.BlockSpec((1, d), lambda i: (0, 0)),
            pl.BlockSpec((d, d), lambda i: (0, 0)),
        ],
        out_specs=pl.BlockSpec((_ROWS, d), lambda i: (i, 0)),
        out_shape=jax.ShapeDtypeStruct((n, d), jnp.float32),
    )(p, hp, dis, b1, w2)


def _tc_c(q, hp2, dis, b2, n, d):
    grid = (n // _ROWS,)
    return pl.pallas_call(
        _tc_c_body,
        grid=grid,
        in_specs=[
            pl.BlockSpec((NC, _ROWS, d), lambda i: (0, i, 0)),
            pl.BlockSpec((_ROWS, d), lambda i: (i, 0)),
            pl.BlockSpec((_ROWS, 1), lambda i: (i, 0)),
            pl.BlockSpec((1, d), lambda i: (0, 0)),
        ],
        out_specs=pl.BlockSpec((_ROWS, d), lambda i: (i, 0)),
        out_shape=jax.ShapeDtypeStruct((n, d), jnp.float32),
    )(q, hp2, dis, b2)


def kernel(x, edge_index, edge_weight, W1, b1, W2, b2):
    n, d = x.shape
    e = edge_weight.shape[0]
    src = edge_index[0]
    dst = edge_index[1]
    ew = edge_weight
    b1r = b1.reshape(1, d)
    b2r = b2.reshape(1, d)

    deg_kernel = _make_deg_kernel(n, e)
    agg_kernel = _make_agg_kernel(n, e, d)
    zeros = jnp.zeros((n, d), jnp.float32)

    parts_t = deg_kernel(dst, ew).reshape(NW, n).T
    dis, hp1 = _tc_a(parts_t, x, W1, n, d)
    p1 = agg_kernel(hp1, src, dst, ew, zeros)
    hp2 = _tc_b(p1, hp1, dis, b1r, W2, n, d)
    p2 = agg_kernel(hp2, src, dst, ew, zeros)
    out = _tc_c(p2, hp2, dis, b2r, n, d)
    return out


# speedup vs baseline: 24.1377x; 2.4500x over previous
import functools

import jax
import jax.numpy as jnp
from jax import lax
from jax.experimental import pallas as pl
from jax.experimental.pallas import tpu as pltpu
from jax.experimental.pallas import tpu_sc as plsc

NC = 2
NS = 16
NW = NC * NS
LANES = 16


def _make_deg_kernel(n, e):
    assert e % NW == 0
    e_per = e // NW
    assert e_per % LANES == 0 and n % LANES == 0
    mesh = plsc.VectorSubcoreMesh(core_axis_name="c", subcore_axis_name="s")

    @functools.partial(
        pl.kernel,
        mesh=mesh,
        compiler_params=pltpu.CompilerParams(needs_layout_passes=False),
        out_type=jax.ShapeDtypeStruct((NW * n,), jnp.float32),
        scratch_types=[
            pltpu.VMEM((e_per,), jnp.int32),
            pltpu.VMEM((e_per,), jnp.float32),
            pltpu.VMEM((n,), jnp.float32),
        ],
    )
    def deg_kernel(dst_hbm, ew_hbm, out_hbm, dst_v, ew_v, deg_v):
        c = lax.axis_index("c")
        s = lax.axis_index("s")
        wid = s * NC + c
        base = pl.multiple_of(wid * e_per, 8)
        pltpu.sync_copy(dst_hbm.at[pl.ds(base, e_per)], dst_v)
        pltpu.sync_copy(ew_hbm.at[pl.ds(base, e_per)], ew_v)

        zeros = jnp.zeros((LANES,), jnp.float32)

        def zero_body(i, carry):
            deg_v[pl.ds(i * LANES, LANES)] = zeros
            return carry

        lax.fori_loop(0, n // LANES, zero_body, 0)

        def edge_body(g, carry):
            idx = dst_v[pl.ds(g * LANES, LANES)]
            w = ew_v[pl.ds(g * LANES, LANES)]
            plsc.addupdate_scatter(deg_v, [idx], w)
            return carry

        lax.fori_loop(0, e_per // LANES, edge_body, 0)
        out0 = pl.multiple_of(wid * n, 8)
        pltpu.sync_copy(deg_v, out_hbm.at[pl.ds(out0, n)])

    return deg_kernel


_CHUNK = 40
_NBUF = 5


def _make_agg_kernel(n, e, d):
    assert e % NW == 0
    e_per = e // NW
    assert e_per % _CHUNK == 0
    n_chunks = e_per // _CHUNK
    assert n_chunks % (2 * _NBUF) == 0
    n_outer = n_chunks // _NBUF
    mesh = plsc.VectorSubcoreMesh(core_axis_name="c", subcore_axis_name="s")

    @functools.partial(
        pl.kernel,
        mesh=mesh,
        compiler_params=pltpu.CompilerParams(needs_layout_passes=False),
        out_type=jax.ShapeDtypeStruct((NC, n, d), jnp.float32),
        scratch_types=[
            [pltpu.VMEM((_CHUNK, d), jnp.float32)] * _NBUF,
            [pltpu.VMEM((3, _CHUNK), jnp.int32)] * (2 * _NBUF),
            [pltpu.SemaphoreType.DMA] * _NBUF,
            [pltpu.SemaphoreType.DMA] * _NBUF,
            [pltpu.SemaphoreType.DMA] * (2 * _NBUF),
            pltpu.VMEM_SHARED((n, d), jnp.float32),
        ],
    )
    def agg_kernel(hp_hbm, ed_hbm, zeros_hbm, out_hbm,
                   bufs, ibufs, gsems, ssems, isems, acc_sh):
        c = lax.axis_index("c")
        s = lax.axis_index("s")
        wid = s * NC + c

        for b in range(_NBUF):
            pltpu.async_copy(ed_hbm.at[wid, b], ibufs[b], isems[b])

        @pl.when(s == 0)
        def _():
            pltpu.sync_copy(zeros_hbm, acc_sh)

        plsc.subcore_barrier()

        def drain_rows(sem):
            pltpu.make_async_copy(hp_hbm.at[pl.ds(0, _CHUNK)], bufs[0],
                                  sem).wait()

        def drain_idx(sem):
            pltpu.make_async_copy(ed_hbm.at[0, 0], ibufs[0], sem).wait()

        def scale(ib, buf):
            two = jnp.full((LANES,), 2, jnp.int32)

            @plsc.parallel_loop(0, _CHUNK, unroll=2)
            def _(i):
                wi = plsc.load_gather(ib, [two, jnp.full((LANES,), i, jnp.int32)])
                w = plsc.bitcast(wi, jnp.float32)
                for j in range(d // LANES):
                    sl = pl.ds(j * LANES, LANES)
                    buf[i, sl] = buf[i, sl] * w

        def half(o, par):
            for b in range(_NBUF):
                g = o * _NBUF + b
                sl = b + _NBUF * par
                nsl = b + _NBUF * (1 - par)

                @pl.when(o > 0)
                def _():
                    drain_rows(ssems[b])
                drain_idx(isems[sl])
                pltpu.async_copy(hp_hbm.at[ibufs[sl].at[0]], bufs[b], gsems[b])

                @pl.when(o < n_outer - 1)
                def _():
                    pltpu.async_copy(ed_hbm.at[wid, g + _NBUF], ibufs[nsl],
                                     isems[nsl])
            for b in range(_NBUF):
                sl = b + _NBUF * par
                drain_rows(gsems[b])
                scale(ibufs[sl], bufs[b])
                pltpu.async_copy(bufs[b], acc_sh.at[ibufs[sl].at[1]], ssems[b],
                                 add=True)

        @pl.loop(0, n_outer // 2)
        def _(q):
            half(2 * q, 0)
            half(2 * q + 1, 1)

        for b in range(_NBUF):
            drain_rows(ssems[b])
        plsc.subcore_barrier()

        @pl.when(s == 0)
        def _():
            pltpu.sync_copy(acc_sh, out_hbm.at[c])

    return agg_kernel


_ROWS = 1000


def _tc_a_body(parts_ref, x_ref, w_ref, dis_ref, hp_ref):
    deg = jnp.sum(parts_ref[...], axis=1, keepdims=True) + 1.0
    dis = lax.rsqrt(deg)
    dis_ref[...] = dis
    h = jnp.dot(x_ref[...], w_ref[...], preferred_element_type=jnp.float32)
    hp_ref[...] = dis * h


def _tc_b_body(p_ref, hp_ref, dis_ref, b_ref, w_ref, hp2_ref):
    dis = dis_ref[...]
    agg = p_ref[0] + p_ref[1] + hp_ref[...]
    z = jnp.maximum(dis * agg + b_ref[...], 0.0)
    h2 = jnp.dot(z, w_ref[...], preferred_element_type=jnp.float32)
    hp2_ref[...] = dis * h2


def _tc_c_body(q_ref, hp2_ref, dis_ref, b_ref, out_ref):
    dis = dis_ref[...]
    out_ref[...] = dis * (q_ref[0] + q_ref[1] + hp2_ref[...]) + b_ref[...]


def _tc_a(parts, x, w1, n, d):
    grid = (n // _ROWS,)
    return pl.pallas_call(
        _tc_a_body,
        grid=grid,
        in_specs=[
            pl.BlockSpec((_ROWS, NW), lambda i: (i, 0)),
            pl.BlockSpec((_ROWS, d), lambda i: (i, 0)),
            pl.BlockSpec((d, d), lambda i: (0, 0)),
        ],
        out_specs=[
            pl.BlockSpec((_ROWS, 1), lambda i: (i, 0)),
            pl.BlockSpec((_ROWS, d), lambda i: (i, 0)),
        ],
        out_shape=[
            jax.ShapeDtypeStruct((n, 1), jnp.float32),
            jax.ShapeDtypeStruct((n, d), jnp.float32),
        ],
    )(parts, x, w1)


def _tc_b(p, hp, dis, b1, w2, n, d):
    grid = (n // _ROWS,)
    return pl.pallas_call(
        _tc_b_body,
        grid=grid,
        in_specs=[
            pl.BlockSpec((NC, _ROWS, d), lambda i: (0, i, 0)),
            pl.BlockSpec((_ROWS, d), lambda i: (i, 0)),
            pl.BlockSpec((_ROWS, 1), lambda i: (i, 0)),
            pl.BlockSpec((1, d), lambda i: (0, 0)),
            pl.BlockSpec((d, d), lambda i: (0, 0)),
        ],
        out_specs=pl.BlockSpec((_ROWS, d), lambda i: (i, 0)),
        out_shape=jax.ShapeDtypeStruct((n, d), jnp.float32),
    )(p, hp, dis, b1, w2)


def _tc_c(q, hp2, dis, b2, n, d):
    grid = (n // _ROWS,)
    return pl.pallas_call(
        _tc_c_body,
        grid=grid,
        in_specs=[
            pl.BlockSpec((NC, _ROWS, d), lambda i: (0, i, 0)),
            pl.BlockSpec((_ROWS, d), lambda i: (i, 0)),
            pl.BlockSpec((_ROWS, 1), lambda i: (i, 0)),
            pl.BlockSpec((1, d), lambda i: (0, 0)),
        ],
        out_specs=pl.BlockSpec((_ROWS, d), lambda i: (i, 0)),
        out_shape=jax.ShapeDtypeStruct((n, d), jnp.float32),
    )(q, hp2, dis, b2)


def kernel(x, edge_index, edge_weight, W1, b1, W2, b2):
    n, d = x.shape
    e = edge_weight.shape[0]
    src = edge_index[0]
    dst = edge_index[1]
    ew = edge_weight
    b1r = b1.reshape(1, d)
    b2r = b2.reshape(1, d)

    deg_kernel = _make_deg_kernel(n, e)
    agg_kernel = _make_agg_kernel(n, e, d)
    zeros = jnp.zeros((n, d), jnp.float32)
    e_per = e // NW
    n_chunks = e_per // _CHUNK
    ewi = jax.lax.bitcast_convert_type(ew, jnp.int32)
    ed4 = jnp.stack(
        [a.reshape(NW, n_chunks, _CHUNK) for a in (src, dst, ewi)], axis=2)

    parts_t = deg_kernel(dst, ew).reshape(NW, n).T
    dis, hp1 = _tc_a(parts_t, x, W1, n, d)
    p1 = agg_kernel(hp1, ed4, zeros)
    hp2 = _tc_b(p1, hp1, dis, b1r, W2, n, d)
    p2 = agg_kernel(hp2, ed4, zeros)
    out = _tc_c(p2, hp2, dis, b2r, n, d)
    return out


# speedup vs baseline: 25.8689x; 1.0717x over previous
import functools

import jax
import jax.numpy as jnp
from jax import lax
from jax.experimental import pallas as pl
from jax.experimental.pallas import tpu as pltpu
from jax.experimental.pallas import tpu_sc as plsc

NC = 2
NS = 16
NW = NC * NS
LANES = 16


def _make_deg_kernel(n, e):
    assert e % NW == 0
    e_per = e // NW
    assert e_per % LANES == 0 and n % LANES == 0
    mesh = plsc.VectorSubcoreMesh(core_axis_name="c", subcore_axis_name="s")

    @functools.partial(
        pl.kernel,
        mesh=mesh,
        compiler_params=pltpu.CompilerParams(needs_layout_passes=False),
        out_type=jax.ShapeDtypeStruct((NW * n,), jnp.float32),
        scratch_types=[
            pltpu.VMEM((e_per,), jnp.int32),
            pltpu.VMEM((e_per,), jnp.float32),
            pltpu.VMEM((n,), jnp.float32),
        ],
    )
    def deg_kernel(dst_hbm, ew_hbm, out_hbm, dst_v, ew_v, deg_v):
        c = lax.axis_index("c")
        s = lax.axis_index("s")
        wid = s * NC + c
        base = pl.multiple_of(wid * e_per, 8)
        pltpu.sync_copy(dst_hbm.at[pl.ds(base, e_per)], dst_v)
        pltpu.sync_copy(ew_hbm.at[pl.ds(base, e_per)], ew_v)

        zeros = jnp.zeros((LANES,), jnp.float32)

        def zero_body(i, carry):
            deg_v[pl.ds(i * LANES, LANES)] = zeros
            return carry

        lax.fori_loop(0, n // LANES, zero_body, 0)

        def edge_body(g, carry):
            idx = dst_v[pl.ds(g * LANES, LANES)]
            w = ew_v[pl.ds(g * LANES, LANES)]
            plsc.addupdate_scatter(deg_v, [idx], w)
            return carry

        lax.fori_loop(0, e_per // LANES, edge_body, 0)
        out0 = pl.multiple_of(wid * n, 8)
        pltpu.sync_copy(deg_v, out_hbm.at[pl.ds(out0, n)])

    return deg_kernel


_CHUNK = 50
_NBUF = 5


def _make_agg_kernel(n, e, d):
    assert e % NW == 0
    e_per = e // NW
    assert e_per % _CHUNK == 0
    n_chunks = e_per // _CHUNK
    assert n_chunks % (2 * _NBUF) == 0
    n_outer = n_chunks // _NBUF
    mesh = plsc.VectorSubcoreMesh(core_axis_name="c", subcore_axis_name="s")

    @functools.partial(
        pl.kernel,
        mesh=mesh,
        compiler_params=pltpu.CompilerParams(needs_layout_passes=False),
        out_type=jax.ShapeDtypeStruct((NC, n, d), jnp.float32),
        scratch_types=[
            [pltpu.VMEM((_CHUNK, d), jnp.float32)] * _NBUF,
            [pltpu.VMEM((3, _CHUNK), jnp.int32)] * (2 * _NBUF),
            [pltpu.SemaphoreType.DMA] * _NBUF,
            [pltpu.SemaphoreType.DMA] * _NBUF,
            [pltpu.SemaphoreType.DMA] * (2 * _NBUF),
            pltpu.VMEM_SHARED((n, d), jnp.float32),
        ],
    )
    def agg_kernel(hp_hbm, ed_hbm, zeros_hbm, dummy_hbm, out_hbm,
                   bufs, ibufs, gsems, ssems, isems, acc_sh):
        c = lax.axis_index("c")
        s = lax.axis_index("s")
        wid = s * NC + c

        for b in range(_NBUF):
            pltpu.async_copy(ed_hbm.at[wid, b], ibufs[b], isems[b])

        @pl.when(s == 0)
        def _():
            pltpu.sync_copy(zeros_hbm, acc_sh)

        plsc.subcore_barrier()

        def drain_rows(sem):
            pltpu.make_async_copy(dummy_hbm, bufs[0], sem).wait()

        def drain_idx(sem):
            pltpu.make_async_copy(ed_hbm.at[0, 0], ibufs[0], sem).wait()

        def scale(ib, buf):
            two = jnp.full((LANES,), 2, jnp.int32)

            @plsc.parallel_loop(0, _CHUNK, unroll=2)
            def _(i):
                wi = plsc.load_gather(ib, [two, jnp.full((LANES,), i, jnp.int32)])
                w = plsc.bitcast(wi, jnp.float32)
                for j in range(d // LANES):
                    sl = pl.ds(j * LANES, LANES)
                    buf[i, sl] = buf[i, sl] * w

        def half(o, par):
            for b in range(_NBUF):
                g = o * _NBUF + b
                sl = b + _NBUF * par
                nsl = b + _NBUF * (1 - par)

                @pl.when(o > 0)
                def _():
                    drain_rows(ssems[b])
                drain_idx(isems[sl])
                pltpu.async_copy(hp_hbm.at[ibufs[sl].at[0]], bufs[b], gsems[b])

                @pl.when(o < n_outer - 1)
                def _():
                    pltpu.async_copy(ed_hbm.at[wid, g + _NBUF], ibufs[nsl],
                                     isems[nsl])
            for b in range(_NBUF):
                sl = b + _NBUF * par
                drain_rows(gsems[b])
                scale(ibufs[sl], bufs[b])
                pltpu.async_copy(bufs[b], acc_sh.at[ibufs[sl].at[1]], ssems[b],
                                 add=True)

        @pl.loop(0, n_outer // 2)
        def _(q):
            half(2 * q, 0)
            half(2 * q + 1, 1)

        for b in range(_NBUF):
            drain_rows(ssems[b])
        plsc.subcore_barrier()

        @pl.when(s == 0)
        def _():
            pltpu.sync_copy(acc_sh, out_hbm.at[c])

    return agg_kernel


_ROWS = 1000


def _tc_a_body(parts_ref, x_ref, w_ref, dis_ref, hp_ref):
    deg = jnp.sum(parts_ref[...], axis=1, keepdims=True) + 1.0
    dis = lax.rsqrt(deg)
    dis_ref[...] = dis
    h = jnp.dot(x_ref[...], w_ref[...], preferred_element_type=jnp.float32)
    hp_ref[...] = dis * h


def _tc_b_body(p_ref, hp_ref, dis_ref, b_ref, w_ref, hp2_ref):
    dis = dis_ref[...]
    agg = p_ref[0] + p_ref[1] + hp_ref[...]
    z = jnp.maximum(dis * agg + b_ref[...], 0.0)
    h2 = jnp.dot(z, w_ref[...], preferred_element_type=jnp.float32)
    hp2_ref[...] = dis * h2


def _tc_c_body(q_ref, hp2_ref, dis_ref, b_ref, out_ref):
    dis = dis_ref[...]
    out_ref[...] = dis * (q_ref[0] + q_ref[1] + hp2_ref[...]) + b_ref[...]


def _tc_a(parts, x, w1, n, d):
    grid = (n // _ROWS,)
    return pl.pallas_call(
        _tc_a_body,
        grid=grid,
        in_specs=[
            pl.BlockSpec((_ROWS, NW), lambda i: (i, 0)),
            pl.BlockSpec((_ROWS, d), lambda i: (i, 0)),
            pl.BlockSpec((d, d), lambda i: (0, 0)),
        ],
        out_specs=[
            pl.BlockSpec((_ROWS, 1), lambda i: (i, 0)),
            pl.BlockSpec((_ROWS, d), lambda i: (i, 0)),
        ],
        out_shape=[
            jax.ShapeDtypeStruct((n, 1), jnp.float32),
            jax.ShapeDtypeStruct((n, d), jnp.float32),
        ],
    )(parts, x, w1)


def _tc_b(p, hp, dis, b1, w2, n, d):
    grid = (n // _ROWS,)
    return pl.pallas_call(
        _tc_b_body,
        grid=grid,
        in_specs=[
            pl.BlockSpec((NC, _ROWS, d), lambda i: (0, i, 0)),
            pl.BlockSpec((_ROWS, d), lambda i: (i, 0)),
            pl.BlockSpec((_ROWS, 1), lambda i: (i, 0)),
            pl.BlockSpec((1, d), lambda i: (0, 0)),
            pl.BlockSpec((d, d), lambda i: (0, 0)),
        ],
        out_specs=pl.BlockSpec((_ROWS, d), lambda i: (i, 0)),
        out_shape=jax.ShapeDtypeStruct((n, d), jnp.float32),
    )(p, hp, dis, b1, w2)


def _tc_c(q, hp2, dis, b2, n, d):
    grid = (n // _ROWS,)
    return pl.pallas_call(
        _tc_c_body,
        grid=grid,
        in_specs=[
            pl.BlockSpec((NC, _ROWS, d), lambda i: (0, i, 0)),
            pl.BlockSpec((_ROWS, d), lambda i: (i, 0)),
            pl.BlockSpec((_ROWS, 1), lambda i: (i, 0)),
            pl.BlockSpec((1, d), lambda i: (0, 0)),
        ],
        out_specs=pl.BlockSpec((_ROWS, d), lambda i: (i, 0)),
        out_shape=jax.ShapeDtypeStruct((n, d), jnp.float32),
    )(q, hp2, dis, b2)


def kernel(x, edge_index, edge_weight, W1, b1, W2, b2):
    n, d = x.shape
    e = edge_weight.shape[0]
    src = edge_index[0]
    dst = edge_index[1]
    ew = edge_weight
    b1r = b1.reshape(1, d)
    b2r = b2.reshape(1, d)

    deg_kernel = _make_deg_kernel(n, e)
    agg_kernel = _make_agg_kernel(n, e, d)
    zeros = jnp.zeros((n, d), jnp.float32)
    e_per = e // NW
    n_chunks = e_per // _CHUNK
    ewi = jax.lax.bitcast_convert_type(ew, jnp.int32)
    ed4 = jnp.stack(
        [a.reshape(NW, n_chunks, _CHUNK) for a in (src, dst, ewi)], axis=2)
    dummy = jnp.zeros((_CHUNK, d), jnp.float32)

    parts_t = deg_kernel(dst, ew).reshape(NW, n).T
    dis, hp1 = _tc_a(parts_t, x, W1, n, d)
    p1 = agg_kernel(hp1, ed4, zeros, dummy)
    hp2 = _tc_b(p1, hp1, dis, b1r, W2, n, d)
    p2 = agg_kernel(hp2, ed4, zeros, dummy)
    out = _tc_c(p2, hp2, dis, b2r, n, d)
    return out
